# Initial kernel scaffold; baseline (speedup 1.0000x reference)
import jax
import jax.numpy as jnp
from jax.experimental import pallas as pl


def kernel(x, edge_index, graph_input, batch, params):
    raise NotImplementedError("write your pallas kernel here")


# speedup vs baseline: 2.7448x; 2.7448x over previous
import functools

import jax
import jax.numpy as jnp
from jax import lax
from jax.experimental import pallas as pl
from jax.experimental.pallas import tpu as pltpu
from jax.experimental.pallas import tpu_sc as plsc

B = 128
NPG = 128
N = B * NPG
E = 4 * N
K = 4
HEADS = 8
DIN = 16
DG = 8
H = 256
NCLS = 4
EPS = 1e-5
F32 = jnp.float32


def _leaky(v):
    return jnp.where(v >= 0, v, 0.01 * v)


def _dot(a, b):
    return jax.lax.dot_general(a, b, (((1,), (0,)), ((), ())),
                               preferred_element_type=F32,
                               precision=jax.lax.Precision.HIGHEST)


def _dot_nt(a, b):
    return jax.lax.dot_general(a, b, (((1,), (1,)), ((), ())),
                               preferred_element_type=F32,
                               precision=jax.lax.Precision.HIGHEST)


def _graphnorm_body(x_ref, gp_ref, o_ref):
    x3 = x_ref[...].reshape(8, NPG, DIN)
    w = gp_ref[0:1, :].reshape(1, 1, DIN)
    b = gp_ref[1:2, :].reshape(1, 1, DIN)
    a = gp_ref[2:3, :].reshape(1, 1, DIN)
    m = jnp.mean(x3, axis=1, keepdims=True)
    o = x3 - a * m
    var = jnp.mean(o * o, axis=1, keepdims=True)
    y = w * o * jax.lax.rsqrt(var + EPS) + b
    o_ref[...] = y.reshape(8 * NPG, DIN)


def _graphnorm(x, gp):
    return pl.pallas_call(
        _graphnorm_body,
        grid=(B // 8,),
        in_specs=[
            pl.BlockSpec((8 * NPG, DIN), lambda g: (g, 0)),
            pl.BlockSpec((3, DIN), lambda g: (0, 0)),
        ],
        out_specs=pl.BlockSpec((8 * NPG, DIN), lambda g: (g, 0)),
        out_shape=jax.ShapeDtypeStruct((N, DIN), F32),
    )(x, gp)


def _sc_gather(xf, src, dst):
    mesh = plsc.VectorSubcoreMesh(core_axis_name="c", subcore_axis_name="s")
    n_workers = 32
    per_w = E // n_workers
    chunk = 128

    @functools.partial(
        pl.kernel,
        mesh=mesh,
        compiler_params=pltpu.CompilerParams(use_tc_tiling_on_sc=False),
        out_type=[
            jax.ShapeDtypeStruct((E, DIN), F32),
            jax.ShapeDtypeStruct((E, DIN), F32),
        ],
        scratch_types=[
            pltpu.VMEM((chunk,), jnp.int32),
            pltpu.VMEM((chunk,), jnp.int32),
            pltpu.VMEM((chunk, DIN), F32),
            pltpu.VMEM((chunk, DIN), F32),
        ],
    )
    def gk(xf_hbm, src_hbm, dst_hbm, xi_hbm, xj_hbm, idxd_v, idxs_v,
           rowd_v, rows_v):
        wid = lax.axis_index("s") * 2 + lax.axis_index("c")
        base = wid * per_w

        @pl.loop(0, per_w, step=chunk)
        def _(i):
            e0 = base + i
            pltpu.sync_copy(dst_hbm.at[pl.ds(e0, chunk)], idxd_v)
            pltpu.sync_copy(src_hbm.at[pl.ds(e0, chunk)], idxs_v)
            pltpu.sync_copy(xf_hbm.at[idxd_v], rowd_v)
            pltpu.sync_copy(xf_hbm.at[idxs_v], rows_v)
            pltpu.sync_copy(rowd_v, xi_hbm.at[pl.ds(e0, chunk)])
            pltpu.sync_copy(rows_v, xj_hbm.at[pl.ds(e0, chunk)])

    return gk(xf, src, dst)


def _edge_mlp_tail(h1, w2_ref, w3_ref, vp_ref):
    b2 = vp_ref[3:4, :]
    s2 = vp_ref[4:5, :]
    be2 = vp_ref[5:6, :]
    b3 = vp_ref[6:7, :]
    s3 = vp_ref[7:8, :]
    be3 = vp_ref[8:9, :]
    h2 = _leaky(_dot(h1, w2_ref[...]) + b2) * s2 + be2
    h3 = _leaky(_dot(h2, w3_ref[...]) + b3) * s3 + be3
    return h3


def _conv1_body(xi_ref, xj_ref, cd_ref, cb_ref, w2_ref, w3_ref, vp_ref,
                o_ref):
    b1 = vp_ref[0:1, :]
    s1 = vp_ref[1:2, :]
    be1 = vp_ref[2:3, :]
    u = _dot(xi_ref[...], cd_ref[...]) + _dot(xj_ref[...], cb_ref[...]) + b1
    h1 = _leaky(u) * s1 + be1
    o_ref[...] = _edge_mlp_tail(h1, w2_ref, w3_ref, vp_ref)


def _conv1_mlp(xi_g, xj_g, cd, cb, w2, w3, vp):
    blk = 4096
    return pl.pallas_call(
        _conv1_body,
        grid=(E // blk,),
        in_specs=[
            pl.BlockSpec((blk, DIN), lambda g: (g, 0)),
            pl.BlockSpec((blk, DIN), lambda g: (g, 0)),
            pl.BlockSpec((DIN, H), lambda g: (0, 0)),
            pl.BlockSpec((DIN, H), lambda g: (0, 0)),
            pl.BlockSpec((H, H), lambda g: (0, 0)),
            pl.BlockSpec((H, H), lambda g: (0, 0)),
            pl.BlockSpec((16, H), lambda g: (0, 0)),
        ],
        out_specs=pl.BlockSpec((blk, H), lambda g: (g, 0)),
        out_shape=jax.ShapeDtypeStruct((E, H), F32),
    )(xi_g, xj_g, cd, cb, w2, w3, vp)


def _sc_scatter(msg, dst):
    mesh = plsc.VectorSubcoreMesh(core_axis_name="c", subcore_axis_name="s")
    CW = 64
    chunk = 128
    rows_per_sub = N // 16
    edges_per_sub = E // 16

    @functools.partial(
        pl.kernel,
        mesh=mesh,
        compiler_params=pltpu.CompilerParams(use_tc_tiling_on_sc=False),
        out_type=[
            jax.ShapeDtypeStruct((N, H), F32),
            jax.ShapeDtypeStruct((N, 16), F32),
        ],
        scratch_types=[
            pltpu.VMEM((chunk,), jnp.int32),
            pltpu.VMEM((chunk, CW), F32),
            pltpu.VMEM((chunk, 16), F32),
            pltpu.VMEM((chunk, 16), F32),
            pltpu.VMEM_SHARED((N, CW), F32),
            pltpu.VMEM_SHARED((N, 16), F32),
        ],
    )
    def sk(msg_hbm, dst_hbm, ssum_hbm, cnt_hbm, idx_v, buf_v, ones_v,
           zero16_v, acc_sh, cnt_sh):
        cid = lax.axis_index("c")
        sid = lax.axis_index("s")
        r0 = sid * rows_per_sub
        e_base = sid * edges_per_sub

        @pl.loop(0, chunk)
        def _(i):
            ones_v[i, :] = jnp.full((16,), 1.0, F32)
            zero16_v[i, :] = jnp.zeros((16,), F32)

        for p in range(2):
            col0 = cid * 2 * CW + p * CW

            @pl.loop(0, rows_per_sub, step=chunk)
            def _(r):
                for j in range(CW // 16):
                    pltpu.sync_copy(
                        zero16_v,
                        acc_sh.at[pl.ds(r0 + r, chunk), pl.ds(j * 16, 16)])

            if p == 0:
                @pl.when(cid == 0)
                def _():
                    @pl.loop(0, rows_per_sub, step=chunk)
                    def _(r):
                        pltpu.sync_copy(zero16_v,
                                        cnt_sh.at[pl.ds(r0 + r, chunk)])

            plsc.subcore_barrier()

            @pl.loop(0, edges_per_sub, step=chunk)
            def _(i):
                e0 = e_base + i
                pltpu.sync_copy(dst_hbm.at[pl.ds(e0, chunk)], idx_v)
                pltpu.sync_copy(
                    msg_hbm.at[pl.ds(e0, chunk), pl.ds(col0, CW)], buf_v)
                pltpu.sync_copy(buf_v, acc_sh.at[idx_v], add=True)
                if p == 0:
                    @pl.when(cid == 0)
                    def _():
                        pltpu.sync_copy(ones_v, cnt_sh.at[idx_v], add=True)

            plsc.subcore_barrier()

            @pl.loop(0, rows_per_sub, step=chunk)
            def _(r):
                pltpu.sync_copy(
                    acc_sh.at[pl.ds(r0 + r, chunk)],
                    ssum_hbm.at[pl.ds(r0 + r, chunk), pl.ds(col0, CW)])

            if p == 0:
                @pl.when(cid == 0)
                def _():
                    @pl.loop(0, rows_per_sub, step=chunk)
                    def _(r):
                        pltpu.sync_copy(cnt_sh.at[pl.ds(r0 + r, chunk)],
                                        cnt_hbm.at[pl.ds(r0 + r, chunk)])

            plsc.subcore_barrier()

    return sk(msg, dst)


def _combine_body(ssum_ref, cnt_ref, xf_ref, wsc_ref, vp_ref, o_ref):
    bsc = vp_ref[9:10, :]
    ssc = vp_ref[10:11, :]
    besc = vp_ref[11:12, :]
    c = cnt_ref[:, 0:1]
    mean = ssum_ref[...] / jnp.maximum(c, 1.0)
    sc = (_dot(xf_ref[...], wsc_ref[...]) + bsc) * ssc + besc
    o_ref[...] = mean + sc


def _conv1_combine(ssum, cnt, xf, wsc, vp):
    blk = 2048
    return pl.pallas_call(
        _combine_body,
        grid=(N // blk,),
        in_specs=[
            pl.BlockSpec((blk, H), lambda g: (g, 0)),
            pl.BlockSpec((blk, 16), lambda g: (g, 0)),
            pl.BlockSpec((blk, DIN), lambda g: (g, 0)),
            pl.BlockSpec((DIN, H), lambda g: (0, 0)),
            pl.BlockSpec((16, H), lambda g: (0, 0)),
        ],
        out_specs=pl.BlockSpec((blk, H), lambda g: (g, 0)),
        out_shape=jax.ShapeDtypeStruct((N, H), F32),
    )(ssum, cnt, xf, wsc, vp)


def _dyn_body(h_ref, wd_ref, wb_ref, w2_ref, w3_ref, wsc_ref, vp_ref, o_ref):
    h = h_ref[...]
    s2 = jnp.sum(h * h, axis=1, keepdims=True)
    g = _dot_nt(h, h)
    d = s2 + s2.reshape(1, NPG) - 2.0 * g
    rid = lax.broadcasted_iota(jnp.int32, (NPG, NPG), 0)
    cid = lax.broadcasted_iota(jnp.int32, (NPG, NPG), 1)
    d = jnp.where(rid == cid, d + 1e10, d)

    ohs = []
    rem = d
    for _ in range(K):
        mn = jnp.min(rem, axis=1, keepdims=True)
        cand = jnp.where(rem == mn, cid, NPG * 2)
        idx = jnp.min(cand, axis=1, keepdims=True)
        sel = cid == idx
        ohs.append(jnp.where(sel, 1.0, 0.0).astype(F32))
        rem = jnp.where(sel, 3e38, rem)
    p_mat = jnp.concatenate(ohs, axis=0)

    xj = _dot(p_mat, h)
    b1 = vp_ref[0:1, :]
    s1 = vp_ref[1:2, :]
    be1 = vp_ref[2:3, :]
    c1 = _dot(h, wd_ref[...])
    c1t = jnp.concatenate([c1, c1, c1, c1], axis=0)
    u = _dot(xj, wb_ref[...]) + c1t + b1
    h1 = _leaky(u) * s1 + be1
    msg = _edge_mlp_tail(h1, w2_ref, w3_ref, vp_ref)
    mean = jnp.mean(msg.reshape(K, NPG, H), axis=0)

    bsc = vp_ref[9:10, :]
    ssc = vp_ref[10:11, :]
    besc = vp_ref[11:12, :]
    sc = (_dot(h, wsc_ref[...]) + bsc) * ssc + besc
    o_ref[...] = mean + sc


def _dyn_conv(h, wd, wb, w2, w3, wsc, vp):
    return pl.pallas_call(
        _dyn_body,
        grid=(B,),
        in_specs=[
            pl.BlockSpec((NPG, H), lambda g: (g, 0)),
            pl.BlockSpec((H, H), lambda g: (0, 0)),
            pl.BlockSpec((H, H), lambda g: (0, 0)),
            pl.BlockSpec((H, H), lambda g: (0, 0)),
            pl.BlockSpec((H, H), lambda g: (0, 0)),
            pl.BlockSpec((H, H), lambda g: (0, 0)),
            pl.BlockSpec((16, H), lambda g: (0, 0)),
        ],
        out_specs=pl.BlockSpec((NPG, H), lambda g: (g, 0)),
        out_shape=jax.ShapeDtypeStruct((N, H), F32),
    )(h, wd, wb, w2, w3, wsc, vp)


ATT_BLK = 512
HD = H // HEADS


def _attn_body(h_ref, wqt_ref, wkt_ref, wvt_ref, vp_ref, o_ref):
    h = h_ref[...]
    ht = jnp.transpose(h)
    bq = vp_ref[0:1, :].reshape(H, 1)
    bk = vp_ref[1:2, :].reshape(H, 1)
    bv = vp_ref[2:3, :].reshape(H, 1)
    qt = _dot(wqt_ref[...], ht) + bq
    kt = _dot(wkt_ref[...], ht) + bk
    vt = _dot(wvt_ref[...], ht) + bv

    scale = 1.0 / (HD ** 0.5)
    rows = []
    for hh in range(HEADS):
        qh = qt[hh * HD:(hh + 1) * HD, :]
        for gg in range(HEADS):
            kg = kt[gg * HD:(gg + 1) * HD, :]
            rows.append(jnp.sum(qh * kg, axis=0, keepdims=True))
    st = jnp.concatenate(rows, axis=0) * scale
    s3 = st.reshape(HEADS, HEADS, ATT_BLK)
    mx = jnp.max(s3, axis=1, keepdims=True)
    ex = jnp.exp(s3 - mx)
    sm = ex / jnp.sum(ex, axis=1, keepdims=True)

    outs = []
    for hh in range(HEADS):
        acc = sm[hh, 0:1, :] * vt[0:HD, :]
        for gg in range(1, HEADS):
            acc = acc + sm[hh, gg:gg + 1, :] * vt[gg * HD:(gg + 1) * HD, :]
        outs.append(acc)
    att_t = jnp.concatenate(outs, axis=0)

    y = h + jnp.transpose(att_t)
    mu = jnp.mean(y, axis=1, keepdims=True)
    o = y - mu
    var = jnp.mean(o * o, axis=1, keepdims=True)
    lng = vp_ref[3:4, :]
    lnb = vp_ref[4:5, :]
    o_ref[...] = o * jax.lax.rsqrt(var + EPS) * lng + lnb


def _attention(h, wqt, wkt, wvt, vp):
    return pl.pallas_call(
        _attn_body,
        grid=(N // ATT_BLK,),
        in_specs=[
            pl.BlockSpec((ATT_BLK, H), lambda g: (g, 0)),
            pl.BlockSpec((H, H), lambda g: (0, 0)),
            pl.BlockSpec((H, H), lambda g: (0, 0)),
            pl.BlockSpec((H, H), lambda g: (0, 0)),
            pl.BlockSpec((8, H), lambda g: (0, 0)),
        ],
        out_specs=pl.BlockSpec((ATT_BLK, H), lambda g: (g, 0)),
        out_shape=jax.ShapeDtypeStruct((N, H), F32),
    )(h, wqt, wkt, wvt, vp)


def _pool_body(y_ref, w1_ref, w2_ref, vp_ref, o_ref):
    y = y_ref[...]
    b1 = vp_ref[0:1, 0:H // 2]
    b2 = vp_ref[1:2, 0:1]
    t = _leaky(_dot(y, w1_ref[...]) + b1)
    aw = _dot(t, w2_ref[...]) + b2
    aw3 = aw.reshape(8, NPG, 1)
    mx = jnp.max(aw3, axis=1, keepdims=True)
    ex = jnp.exp(aw3 - mx)
    sm = ex / jnp.sum(ex, axis=1, keepdims=True)
    y3 = y.reshape(8, NPG, H)
    o_ref[...] = jnp.sum(y3 * sm, axis=1)


def _pool(y, w1, w2, vp):
    return pl.pallas_call(
        _pool_body,
        grid=(B // 8,),
        in_specs=[
            pl.BlockSpec((8 * NPG, H), lambda g: (g, 0)),
            pl.BlockSpec((H, H // 2), lambda g: (0, 0)),
            pl.BlockSpec((H // 2, 1), lambda g: (0, 0)),
            pl.BlockSpec((2, H), lambda g: (0, 0)),
        ],
        out_specs=pl.BlockSpec((8, H), lambda g: (g, 0)),
        out_shape=jax.ShapeDtypeStruct((B, H), F32),
    )(y, w1, w2, vp)


def _head_body(pooled_ref, gi_ref, w1p_ref, w1g_ref, w2_ref, w3_ref, w4_ref,
               wo_ref, vp_ref, o_ref):
    t1 = (_dot(pooled_ref[...], w1p_ref[...])
          + _dot(gi_ref[...], w1g_ref[...]) + vp_ref[0:1, 0:2 * H])
    z1 = _leaky(t1) * vp_ref[1:2, 0:2 * H] + vp_ref[2:3, 0:2 * H]
    z2 = (_leaky(_dot(z1, w2_ref[...]) + vp_ref[3:4, 0:H])
          * vp_ref[4:5, 0:H] + vp_ref[5:6, 0:H])
    z3 = (_leaky(_dot(z2, w3_ref[...]) + vp_ref[6:7, 0:H // 2])
          * vp_ref[7:8, 0:H // 2] + vp_ref[8:9, 0:H // 2])
    z4 = (_leaky(_dot(z3, w4_ref[...]) + vp_ref[9:10, 0:H // 4])
          * vp_ref[10:11, 0:H // 4] + vp_ref[11:12, 0:H // 4])
    o_ref[...] = _dot(z4, wo_ref[...]) + vp_ref[12:13, 0:NCLS]


def _head(pooled, gi, w1p, w1g, w2, w3, w4, wo, vp):
    return pl.pallas_call(
        _head_body,
        grid=(1,),
        in_specs=[
            pl.BlockSpec((B, H), lambda g: (0, 0)),
            pl.BlockSpec((B, DG), lambda g: (0, 0)),
            pl.BlockSpec((H, 2 * H), lambda g: (0, 0)),
            pl.BlockSpec((DG, 2 * H), lambda g: (0, 0)),
            pl.BlockSpec((2 * H, H), lambda g: (0, 0)),
            pl.BlockSpec((H, H // 2), lambda g: (0, 0)),
            pl.BlockSpec((H // 2, H // 4), lambda g: (0, 0)),
            pl.BlockSpec((H // 4, NCLS), lambda g: (0, 0)),
            pl.BlockSpec((13, 2 * H), lambda g: (0, 0)),
        ],
        out_specs=pl.BlockSpec((B, NCLS), lambda g: (0, 0)),
        out_shape=jax.ShapeDtypeStruct((B, NCLS), F32),
    )(pooled, gi, w1p, w1g, w2, w3, w4, wo, vp)


_BN_S = 1.0 / (1.0 + EPS) ** 0.5


def _conv_pack(c, cin):
    w1 = c["l1"]["w"]
    cd = w1[:cin] - w1[cin:]
    cb = w1[cin:]
    vp = jnp.zeros((16, H), F32)
    rows = [
        c["l1"]["b"], c["g1"] * _BN_S, c["be1"],
        c["l2"]["b"], c["g2"] * _BN_S, c["be2"],
        c["l3"]["b"], c["g3"] * _BN_S, c["be3"],
        c["sc"]["b"], c["gsc"] * _BN_S, c["besc"],
    ]
    vp = vp.at[0:12, :].set(jnp.stack(rows))
    return cd, cb, c["l2"]["w"], c["l3"]["w"], c["sc"]["w"], vp


def kernel(x, edge_index, graph_input, batch, params):
    src = edge_index[0]
    dst = edge_index[1]

    gp = jnp.stack([params["gn_w"], params["gn_b"], params["gn_a"]])
    xf = _graphnorm(x, gp)

    cd1, cb1, w21, w31, wsc1, vp1 = _conv_pack(params["conv1"], DIN)
    xi_g, xj_g = _sc_gather(xf, src, dst)
    msg = _conv1_mlp(xi_g, xj_g, cd1, cb1, w21, w31, vp1)
    ssum, cnt = _sc_scatter(msg, dst)
    h = _conv1_combine(ssum, cnt, xf, wsc1, vp1)

    for name in ("conv2", "conv3", "conv4", "conv5"):
        wd, wb, w2, w3, wsc, vp = _conv_pack(params[name], H)
        h = _dyn_conv(h, wd, wb, w2, w3, wsc, vp)

    a = params["attn"]
    avp = jnp.zeros((8, H), F32)
    avp = avp.at[0:5, :].set(jnp.stack([
        a["q"]["b"], a["k"]["b"], a["v"]["b"], a["lng"], a["lnb"]]))
    y = _attention(h, a["q"]["w"].T, a["k"]["w"].T, a["v"]["w"].T, avp)

    pp = params["pool"]
    pvp = jnp.zeros((2, H), F32)
    pvp = pvp.at[0, 0:H // 2].set(pp["l1"]["b"])
    pvp = pvp.at[1, 0:1].set(pp["l2"]["b"])
    pooled = _pool(y, pp["l1"]["w"], pp["l2"]["w"], pvp)

    s0 = jnp.concatenate([params["bn0g"], jnp.zeros((0,), F32)]) * _BN_S
    b0 = params["bn0b"]
    w1 = params["d1"]["w"]
    w1p = s0[:H, None] * w1[:H]
    w1g = s0[H:, None] * w1[H:]
    b1 = b0 @ w1 + params["d1"]["b"]
    hvp = jnp.zeros((13, 2 * H), F32)
    hvp = hvp.at[0, :].set(b1)
    hvp = hvp.at[1, :].set(params["g1"] * _BN_S)
    hvp = hvp.at[2, :].set(params["b1"])
    hvp = hvp.at[3, 0:H].set(params["d2"]["b"])
    hvp = hvp.at[4, 0:H].set(params["g2"] * _BN_S)
    hvp = hvp.at[5, 0:H].set(params["b2"])
    hvp = hvp.at[6, 0:H // 2].set(params["d3"]["b"])
    hvp = hvp.at[7, 0:H // 2].set(params["g3"] * _BN_S)
    hvp = hvp.at[8, 0:H // 2].set(params["b3"])
    hvp = hvp.at[9, 0:H // 4].set(params["d4"]["b"])
    hvp = hvp.at[10, 0:H // 4].set(params["g4"] * _BN_S)
    hvp = hvp.at[11, 0:H // 4].set(params["b4"])
    hvp = hvp.at[12, 0:NCLS].set(params["out"]["b"])
    return _head(pooled, graph_input, w1p, w1g, params["d2"]["w"],
                 params["d3"]["w"], params["d4"]["w"], params["out"]["w"],
                 hvp)


# speedup vs baseline: 5.9321x; 2.1612x over previous
import functools

import jax
import jax.numpy as jnp
from jax import lax
from jax.experimental import pallas as pl
from jax.experimental.pallas import tpu as pltpu
from jax.experimental.pallas import tpu_sc as plsc

B = 128
NPG = 128
N = B * NPG
E = 4 * N
K = 4
HEADS = 8
DIN = 16
DG = 8
H = 256
NCLS = 4
EPS = 1e-5
F32 = jnp.float32


def _leaky(v):
    return jnp.where(v >= 0, v, 0.01 * v)


def _dot(a, b, precision=jax.lax.Precision.DEFAULT):
    return jax.lax.dot_general(a, b, (((1,), (0,)), ((), ())),
                               preferred_element_type=F32,
                               precision=precision)


def _dot_nt(a, b, precision=jax.lax.Precision.DEFAULT):
    return jax.lax.dot_general(a, b, (((1,), (1,)), ((), ())),
                               preferred_element_type=F32,
                               precision=precision)


def _graphnorm_body(x_ref, gp_ref, o_ref):
    x3 = x_ref[...].reshape(8, NPG, DIN)
    w = gp_ref[0:1, :].reshape(1, 1, DIN)
    b = gp_ref[1:2, :].reshape(1, 1, DIN)
    a = gp_ref[2:3, :].reshape(1, 1, DIN)
    m = jnp.mean(x3, axis=1, keepdims=True)
    o = x3 - a * m
    var = jnp.mean(o * o, axis=1, keepdims=True)
    y = w * o * jax.lax.rsqrt(var + EPS) + b
    o_ref[...] = y.reshape(8 * NPG, DIN)


def _graphnorm(x, gp):
    return pl.pallas_call(
        _graphnorm_body,
        grid=(B // 8,),
        in_specs=[
            pl.BlockSpec((8 * NPG, DIN), lambda g: (g, 0)),
            pl.BlockSpec((3, DIN), lambda g: (0, 0)),
        ],
        out_specs=pl.BlockSpec((8 * NPG, DIN), lambda g: (g, 0)),
        out_shape=jax.ShapeDtypeStruct((N, DIN), F32),
    )(x, gp)


def _sc_gather(xf, src, dst):
    mesh = plsc.VectorSubcoreMesh(core_axis_name="c", subcore_axis_name="s")
    n_workers = 32
    per_w = E // n_workers
    chunk = 128

    @functools.partial(
        pl.kernel,
        mesh=mesh,
        compiler_params=pltpu.CompilerParams(use_tc_tiling_on_sc=False),
        out_type=[
            jax.ShapeDtypeStruct((E, DIN), F32),
            jax.ShapeDtypeStruct((E, DIN), F32),
        ],
        scratch_types=[
            pltpu.VMEM((chunk,), jnp.int32),
            pltpu.VMEM((chunk,), jnp.int32),
            pltpu.VMEM((chunk, DIN), F32),
            pltpu.VMEM((chunk, DIN), F32),
        ],
    )
    def gk(xf_hbm, src_hbm, dst_hbm, xi_hbm, xj_hbm, idxd_v, idxs_v,
           rowd_v, rows_v):
        wid = lax.axis_index("s") * 2 + lax.axis_index("c")
        base = wid * per_w

        @pl.loop(0, per_w, step=chunk)
        def _(i):
            e0 = base + i
            pltpu.sync_copy(dst_hbm.at[pl.ds(e0, chunk)], idxd_v)
            pltpu.sync_copy(src_hbm.at[pl.ds(e0, chunk)], idxs_v)
            pltpu.sync_copy(xf_hbm.at[idxd_v], rowd_v)
            pltpu.sync_copy(xf_hbm.at[idxs_v], rows_v)
            pltpu.sync_copy(rowd_v, xi_hbm.at[pl.ds(e0, chunk)])
            pltpu.sync_copy(rows_v, xj_hbm.at[pl.ds(e0, chunk)])

    return gk(xf, src, dst)


def _edge_mlp_tail(h1, w2_ref, w3_ref, vp_ref):
    b2 = vp_ref[3:4, :]
    s2 = vp_ref[4:5, :]
    be2 = vp_ref[5:6, :]
    b3 = vp_ref[6:7, :]
    s3 = vp_ref[7:8, :]
    be3 = vp_ref[8:9, :]
    h2 = _leaky(_dot(h1, w2_ref[...]) + b2) * s2 + be2
    h3 = _leaky(_dot(h2, w3_ref[...]) + b3) * s3 + be3
    return h3


def _conv1_body(xi_ref, xj_ref, cd_ref, cb_ref, w2_ref, w3_ref, vp_ref,
                o_ref):
    b1 = vp_ref[0:1, :]
    s1 = vp_ref[1:2, :]
    be1 = vp_ref[2:3, :]
    u = _dot(xi_ref[...], cd_ref[...]) + _dot(xj_ref[...], cb_ref[...]) + b1
    h1 = _leaky(u) * s1 + be1
    o_ref[...] = _edge_mlp_tail(h1, w2_ref, w3_ref, vp_ref)


def _conv1_mlp(xi_g, xj_g, cd, cb, w2, w3, vp):
    blk = 4096
    return pl.pallas_call(
        _conv1_body,
        grid=(E // blk,),
        in_specs=[
            pl.BlockSpec((blk, DIN), lambda g: (g, 0)),
            pl.BlockSpec((blk, DIN), lambda g: (g, 0)),
            pl.BlockSpec((DIN, H), lambda g: (0, 0)),
            pl.BlockSpec((DIN, H), lambda g: (0, 0)),
            pl.BlockSpec((H, H), lambda g: (0, 0)),
            pl.BlockSpec((H, H), lambda g: (0, 0)),
            pl.BlockSpec((16, H), lambda g: (0, 0)),
        ],
        out_specs=pl.BlockSpec((blk, H), lambda g: (g, 0)),
        out_shape=jax.ShapeDtypeStruct((E, H), F32),
    )(xi_g, xj_g, cd, cb, w2, w3, vp)


def _sc_scatter(msg, dst):
    mesh = plsc.VectorSubcoreMesh(core_axis_name="c", subcore_axis_name="s")
    CW = 64
    chunk = 128
    rows_per_sub = N // 16
    edges_per_sub = E // 16

    @functools.partial(
        pl.kernel,
        mesh=mesh,
        compiler_params=pltpu.CompilerParams(use_tc_tiling_on_sc=False),
        out_type=[
            jax.ShapeDtypeStruct((N, H), F32),
            jax.ShapeDtypeStruct((N, 16), F32),
        ],
        scratch_types=[
            pltpu.VMEM((chunk,), jnp.int32),
            pltpu.VMEM((chunk, CW), F32),
            pltpu.VMEM((chunk, 16), F32),
            pltpu.VMEM((chunk, 16), F32),
            pltpu.VMEM_SHARED((N, CW), F32),
            pltpu.VMEM_SHARED((N, 16), F32),
        ],
    )
    def sk(msg_hbm, dst_hbm, ssum_hbm, cnt_hbm, idx_v, buf_v, ones_v,
           zero16_v, acc_sh, cnt_sh):
        cid = lax.axis_index("c")
        sid = lax.axis_index("s")
        r0 = sid * rows_per_sub
        e_base = sid * edges_per_sub

        @pl.loop(0, chunk)
        def _(i):
            ones_v[i, :] = jnp.full((16,), 1.0, F32)
            zero16_v[i, :] = jnp.zeros((16,), F32)

        for p in range(2):
            col0 = cid * 2 * CW + p * CW

            @pl.loop(0, rows_per_sub, step=chunk)
            def _(r):
                for j in range(CW // 16):
                    pltpu.sync_copy(
                        zero16_v,
                        acc_sh.at[pl.ds(r0 + r, chunk), pl.ds(j * 16, 16)])

            if p == 0:
                @pl.when(cid == 0)
                def _():
                    @pl.loop(0, rows_per_sub, step=chunk)
                    def _(r):
                        pltpu.sync_copy(zero16_v,
                                        cnt_sh.at[pl.ds(r0 + r, chunk)])

            plsc.subcore_barrier()

            @pl.loop(0, edges_per_sub, step=chunk)
            def _(i):
                e0 = e_base + i
                pltpu.sync_copy(dst_hbm.at[pl.ds(e0, chunk)], idx_v)
                pltpu.sync_copy(
                    msg_hbm.at[pl.ds(e0, chunk), pl.ds(col0, CW)], buf_v)
                pltpu.sync_copy(buf_v, acc_sh.at[idx_v], add=True)
                if p == 0:
                    @pl.when(cid == 0)
                    def _():
                        pltpu.sync_copy(ones_v, cnt_sh.at[idx_v], add=True)

            plsc.subcore_barrier()

            @pl.loop(0, rows_per_sub, step=chunk)
            def _(r):
                pltpu.sync_copy(
                    acc_sh.at[pl.ds(r0 + r, chunk)],
                    ssum_hbm.at[pl.ds(r0 + r, chunk), pl.ds(col0, CW)])

            if p == 0:
                @pl.when(cid == 0)
                def _():
                    @pl.loop(0, rows_per_sub, step=chunk)
                    def _(r):
                        pltpu.sync_copy(cnt_sh.at[pl.ds(r0 + r, chunk)],
                                        cnt_hbm.at[pl.ds(r0 + r, chunk)])

            plsc.subcore_barrier()

    return sk(msg, dst)


def _combine_body(ssum_ref, cnt_ref, xf_ref, wsc_ref, vp_ref, o_ref):
    bsc = vp_ref[9:10, :]
    ssc = vp_ref[10:11, :]
    besc = vp_ref[11:12, :]
    c = cnt_ref[:, 0:1]
    mean = ssum_ref[...] / jnp.maximum(c, 1.0)
    sc = (_dot(xf_ref[...], wsc_ref[...]) + bsc) * ssc + besc
    o_ref[...] = mean + sc


def _conv1_combine(ssum, cnt, xf, wsc, vp):
    blk = 2048
    return pl.pallas_call(
        _combine_body,
        grid=(N // blk,),
        in_specs=[
            pl.BlockSpec((blk, H), lambda g: (g, 0)),
            pl.BlockSpec((blk, 16), lambda g: (g, 0)),
            pl.BlockSpec((blk, DIN), lambda g: (g, 0)),
            pl.BlockSpec((DIN, H), lambda g: (0, 0)),
            pl.BlockSpec((16, H), lambda g: (0, 0)),
        ],
        out_specs=pl.BlockSpec((blk, H), lambda g: (g, 0)),
        out_shape=jax.ShapeDtypeStruct((N, H), F32),
    )(ssum, cnt, xf, wsc, vp)


def _dyn_body(h_ref, wd_ref, wb_ref, w2_ref, w3_ref, wsc_ref, vp_ref, o_ref):
    h = h_ref[...]
    s2 = jnp.sum(h * h, axis=1, keepdims=True)
    g = _dot_nt(h, h, precision=jax.lax.Precision.HIGHEST)
    d = s2 + s2.reshape(1, NPG) - 2.0 * g
    rid = lax.broadcasted_iota(jnp.int32, (NPG, NPG), 0)
    cid = lax.broadcasted_iota(jnp.int32, (NPG, NPG), 1)
    d = jnp.where(rid == cid, d + 1e10, d)

    ohs = []
    rem = d
    for _ in range(K):
        mn = jnp.min(rem, axis=1, keepdims=True)
        cand = jnp.where(rem == mn, cid, NPG * 2)
        idx = jnp.min(cand, axis=1, keepdims=True)
        sel = cid == idx
        ohs.append(jnp.where(sel, 1.0, 0.0).astype(F32))
        rem = jnp.where(sel, 3e38, rem)
    p_mat = jnp.concatenate(ohs, axis=0)

    xj = _dot(p_mat, h)
    b1 = vp_ref[0:1, :]
    s1 = vp_ref[1:2, :]
    be1 = vp_ref[2:3, :]
    c1 = _dot(h, wd_ref[...])
    c1t = jnp.concatenate([c1, c1, c1, c1], axis=0)
    u = _dot(xj, wb_ref[...]) + c1t + b1
    h1 = _leaky(u) * s1 + be1
    msg = _edge_mlp_tail(h1, w2_ref, w3_ref, vp_ref)
    mean = jnp.mean(msg.reshape(K, NPG, H), axis=0)

    bsc = vp_ref[9:10, :]
    ssc = vp_ref[10:11, :]
    besc = vp_ref[11:12, :]
    sc = (_dot(h, wsc_ref[...]) + bsc) * ssc + besc
    o_ref[...] = mean + sc


def _dyn_conv(h, wd, wb, w2, w3, wsc, vp):
    return pl.pallas_call(
        _dyn_body,
        grid=(B,),
        in_specs=[
            pl.BlockSpec((NPG, H), lambda g: (g, 0)),
            pl.BlockSpec((H, H), lambda g: (0, 0)),
            pl.BlockSpec((H, H), lambda g: (0, 0)),
            pl.BlockSpec((H, H), lambda g: (0, 0)),
            pl.BlockSpec((H, H), lambda g: (0, 0)),
            pl.BlockSpec((H, H), lambda g: (0, 0)),
            pl.BlockSpec((16, H), lambda g: (0, 0)),
        ],
        out_specs=pl.BlockSpec((NPG, H), lambda g: (g, 0)),
        out_shape=jax.ShapeDtypeStruct((N, H), F32),
    )(h, wd, wb, w2, w3, wsc, vp)


ATT_BLK = 512
HD = H // HEADS


def _attn_body(h_ref, wqt_ref, wkt_ref, wvt_ref, vp_ref, o_ref):
    h = h_ref[...]
    ht = jnp.transpose(h)
    bq = vp_ref[0:1, :].reshape(H, 1)
    bk = vp_ref[1:2, :].reshape(H, 1)
    bv = vp_ref[2:3, :].reshape(H, 1)
    qt = _dot(wqt_ref[...], ht) + bq
    kt = _dot(wkt_ref[...], ht) + bk
    vt = _dot(wvt_ref[...], ht) + bv

    scale = 1.0 / (HD ** 0.5)
    rows = []
    for hh in range(HEADS):
        qh = qt[hh * HD:(hh + 1) * HD, :]
        for gg in range(HEADS):
            kg = kt[gg * HD:(gg + 1) * HD, :]
            rows.append(jnp.sum(qh * kg, axis=0, keepdims=True))
    st = jnp.concatenate(rows, axis=0) * scale
    s3 = st.reshape(HEADS, HEADS, ATT_BLK)
    mx = jnp.max(s3, axis=1, keepdims=True)
    ex = jnp.exp(s3 - mx)
    sm = ex / jnp.sum(ex, axis=1, keepdims=True)

    outs = []
    for hh in range(HEADS):
        acc = sm[hh, 0:1, :] * vt[0:HD, :]
        for gg in range(1, HEADS):
            acc = acc + sm[hh, gg:gg + 1, :] * vt[gg * HD:(gg + 1) * HD, :]
        outs.append(acc)
    att_t = jnp.concatenate(outs, axis=0)

    y = h + jnp.transpose(att_t)
    mu = jnp.mean(y, axis=1, keepdims=True)
    o = y - mu
    var = jnp.mean(o * o, axis=1, keepdims=True)
    lng = vp_ref[3:4, :]
    lnb = vp_ref[4:5, :]
    o_ref[...] = o * jax.lax.rsqrt(var + EPS) * lng + lnb


def _attention(h, wqt, wkt, wvt, vp):
    return pl.pallas_call(
        _attn_body,
        grid=(N // ATT_BLK,),
        in_specs=[
            pl.BlockSpec((ATT_BLK, H), lambda g: (g, 0)),
            pl.BlockSpec((H, H), lambda g: (0, 0)),
            pl.BlockSpec((H, H), lambda g: (0, 0)),
            pl.BlockSpec((H, H), lambda g: (0, 0)),
            pl.BlockSpec((8, H), lambda g: (0, 0)),
        ],
        out_specs=pl.BlockSpec((ATT_BLK, H), lambda g: (g, 0)),
        out_shape=jax.ShapeDtypeStruct((N, H), F32),
    )(h, wqt, wkt, wvt, vp)


def _pool_body(y_ref, w1_ref, w2_ref, vp_ref, o_ref):
    y = y_ref[...]
    b1 = vp_ref[0:1, 0:H // 2]
    b2 = vp_ref[1:2, 0:1]
    t = _leaky(_dot(y, w1_ref[...]) + b1)
    aw = _dot(t, w2_ref[...]) + b2
    aw3 = aw.reshape(8, NPG, 1)
    mx = jnp.max(aw3, axis=1, keepdims=True)
    ex = jnp.exp(aw3 - mx)
    sm = ex / jnp.sum(ex, axis=1, keepdims=True)
    y3 = y.reshape(8, NPG, H)
    o_ref[...] = jnp.sum(y3 * sm, axis=1)


def _pool(y, w1, w2, vp):
    return pl.pallas_call(
        _pool_body,
        grid=(B // 8,),
        in_specs=[
            pl.BlockSpec((8 * NPG, H), lambda g: (g, 0)),
            pl.BlockSpec((H, H // 2), lambda g: (0, 0)),
            pl.BlockSpec((H // 2, 1), lambda g: (0, 0)),
            pl.BlockSpec((2, H), lambda g: (0, 0)),
        ],
        out_specs=pl.BlockSpec((8, H), lambda g: (g, 0)),
        out_shape=jax.ShapeDtypeStruct((B, H), F32),
    )(y, w1, w2, vp)


def _head_body(pooled_ref, gi_ref, w1p_ref, w1g_ref, w2_ref, w3_ref, w4_ref,
               wo_ref, vp_ref, o_ref):
    t1 = (_dot(pooled_ref[...], w1p_ref[...])
          + _dot(gi_ref[...], w1g_ref[...]) + vp_ref[0:1, 0:2 * H])
    z1 = _leaky(t1) * vp_ref[1:2, 0:2 * H] + vp_ref[2:3, 0:2 * H]
    z2 = (_leaky(_dot(z1, w2_ref[...]) + vp_ref[3:4, 0:H])
          * vp_ref[4:5, 0:H] + vp_ref[5:6, 0:H])
    z3 = (_leaky(_dot(z2, w3_ref[...]) + vp_ref[6:7, 0:H // 2])
          * vp_ref[7:8, 0:H // 2] + vp_ref[8:9, 0:H // 2])
    z4 = (_leaky(_dot(z3, w4_ref[...]) + vp_ref[9:10, 0:H // 4])
          * vp_ref[10:11, 0:H // 4] + vp_ref[11:12, 0:H // 4])
    o_ref[...] = _dot(z4, wo_ref[...]) + vp_ref[12:13, 0:NCLS]


def _head(pooled, gi, w1p, w1g, w2, w3, w4, wo, vp):
    return pl.pallas_call(
        _head_body,
        grid=(1,),
        in_specs=[
            pl.BlockSpec((B, H), lambda g: (0, 0)),
            pl.BlockSpec((B, DG), lambda g: (0, 0)),
            pl.BlockSpec((H, 2 * H), lambda g: (0, 0)),
            pl.BlockSpec((DG, 2 * H), lambda g: (0, 0)),
            pl.BlockSpec((2 * H, H), lambda g: (0, 0)),
            pl.BlockSpec((H, H // 2), lambda g: (0, 0)),
            pl.BlockSpec((H // 2, H // 4), lambda g: (0, 0)),
            pl.BlockSpec((H // 4, NCLS), lambda g: (0, 0)),
            pl.BlockSpec((13, 2 * H), lambda g: (0, 0)),
        ],
        out_specs=pl.BlockSpec((B, NCLS), lambda g: (0, 0)),
        out_shape=jax.ShapeDtypeStruct((B, NCLS), F32),
    )(pooled, gi, w1p, w1g, w2, w3, w4, wo, vp)


_BN_S = 1.0 / (1.0 + EPS) ** 0.5


def _conv_pack(c, cin):
    w1 = c["l1"]["w"]
    cd = w1[:cin] - w1[cin:]
    cb = w1[cin:]
    vp = jnp.zeros((16, H), F32)
    rows = [
        c["l1"]["b"], c["g1"] * _BN_S, c["be1"],
        c["l2"]["b"], c["g2"] * _BN_S, c["be2"],
        c["l3"]["b"], c["g3"] * _BN_S, c["be3"],
        c["sc"]["b"], c["gsc"] * _BN_S, c["besc"],
    ]
    vp = vp.at[0:12, :].set(jnp.stack(rows))
    return cd, cb, c["l2"]["w"], c["l3"]["w"], c["sc"]["w"], vp


def kernel(x, edge_index, graph_input, batch, params):
    src = edge_index[0]
    dst = edge_index[1]

    gp = jnp.stack([params["gn_w"], params["gn_b"], params["gn_a"]])
    xf = _graphnorm(x, gp)

    cd1, cb1, w21, w31, wsc1, vp1 = _conv_pack(params["conv1"], DIN)
    xi_g, xj_g = _sc_gather(xf, src, dst)
    msg = _conv1_mlp(xi_g, xj_g, cd1, cb1, w21, w31, vp1)
    ssum, cnt = _sc_scatter(msg, dst)
    h = _conv1_combine(ssum, cnt, xf, wsc1, vp1)

    for name in ("conv2", "conv3", "conv4", "conv5"):
        wd, wb, w2, w3, wsc, vp = _conv_pack(params[name], H)
        h = _dyn_conv(h, wd, wb, w2, w3, wsc, vp)

    a = params["attn"]
    avp = jnp.zeros((8, H), F32)
    avp = avp.at[0:5, :].set(jnp.stack([
        a["q"]["b"], a["k"]["b"], a["v"]["b"], a["lng"], a["lnb"]]))
    y = _attention(h, a["q"]["w"].T, a["k"]["w"].T, a["v"]["w"].T, avp)

    pp = params["pool"]
    pvp = jnp.zeros((2, H), F32)
    pvp = pvp.at[0, 0:H // 2].set(pp["l1"]["b"])
    pvp = pvp.at[1, 0:1].set(pp["l2"]["b"])
    pooled = _pool(y, pp["l1"]["w"], pp["l2"]["w"], pvp)

    s0 = jnp.concatenate([params["bn0g"], jnp.zeros((0,), F32)]) * _BN_S
    b0 = params["bn0b"]
    w1 = params["d1"]["w"]
    w1p = s0[:H, None] * w1[:H]
    w1g = s0[H:, None] * w1[H:]
    b1 = b0 @ w1 + params["d1"]["b"]
    hvp = jnp.zeros((13, 2 * H), F32)
    hvp = hvp.at[0, :].set(b1)
    hvp = hvp.at[1, :].set(params["g1"] * _BN_S)
    hvp = hvp.at[2, :].set(params["b1"])
    hvp = hvp.at[3, 0:H].set(params["d2"]["b"])
    hvp = hvp.at[4, 0:H].set(params["g2"] * _BN_S)
    hvp = hvp.at[5, 0:H].set(params["b2"])
    hvp = hvp.at[6, 0:H // 2].set(params["d3"]["b"])
    hvp = hvp.at[7, 0:H // 2].set(params["g3"] * _BN_S)
    hvp = hvp.at[8, 0:H // 2].set(params["b3"])
    hvp = hvp.at[9, 0:H // 4].set(params["d4"]["b"])
    hvp = hvp.at[10, 0:H // 4].set(params["g4"] * _BN_S)
    hvp = hvp.at[11, 0:H // 4].set(params["b4"])
    hvp = hvp.at[12, 0:NCLS].set(params["out"]["b"])
    return _head(pooled, graph_input, w1p, w1g, params["d2"]["w"],
                 params["d3"]["w"], params["d4"]["w"], params["out"]["w"],
                 hvp)
